# baseline (speedup 1.0000x reference)
import functools

import jax
import jax.numpy as jnp
from jax import lax
from jax.experimental import pallas as pl
from jax.experimental.pallas import tpu as pltpu
from jax.experimental.pallas import tpu_sc as plsc

D_MODEL = 128
BATCH = 4096
NUM_DIM = 512

_NC = 2
_NS = 16
_NW = _NC * _NS
_BPW = BATCH // _NW

_K = 4
_CHUNK = BATCH // _K


def _gather_sc(i0, i1, i2, i3, t0, t1, t2, t3):
    mesh = plsc.VectorSubcoreMesh(core_axis_name="c", subcore_axis_name="s")

    @functools.partial(
        pl.kernel,
        mesh=mesh,
        out_type=jax.ShapeDtypeStruct((6, BATCH, D_MODEL), jnp.float32),
        scratch_types=[
            pltpu.VMEM((4, _BPW), jnp.int32),
            pltpu.VMEM((4, _BPW, D_MODEL), jnp.float32),
            pltpu.SemaphoreType.DMA((4,)),
            pltpu.SemaphoreType.DMA((4,)),
            pltpu.SemaphoreType.DMA((4,)),
        ],
    )
    def body(ih0, ih1, ih2, ih3, tb0, tb1, tb2, tb3, out_hbm,
             idx_v, rows_v, sem_i, sem_g, sem_w):
        wid = lax.axis_index("s") * _NC + lax.axis_index("c")
        base = wid * _BPW
        iloads = []
        for i, ih in enumerate((ih0, ih1, ih2, ih3)):
            iloads.append(
                pltpu.async_copy(ih.at[pl.ds(base, _BPW)], idx_v.at[i],
                                 sem_i.at[i]))
        gathers = []
        for i, tbl in enumerate((tb0, tb1, tb2, tb3)):
            iloads[i].wait()
            gathers.append(
                pltpu.async_copy(tbl.at[idx_v.at[i]], rows_v.at[i], sem_g.at[i]))
        writes = []
        for i, g in enumerate(gathers):
            g.wait()
            writes.append(
                pltpu.async_copy(rows_v.at[i],
                                 out_hbm.at[i, pl.ds(base, _BPW)], sem_w.at[i]))
        for w in writes:
            w.wait()

    return body(i0, i1, i2, i3, t0, t1, t2, t3)


def _num_body(x0_ref, x1_ref, g0_ref, be0_ref, w0_ref, b0_ref,
              g1_ref, be1_ref, w1_ref, b1_ref, out_ref,
              stash, acc_s, acc_q, scale_r, shift_r):
    j = pl.program_id(0)
    k = pl.program_id(1)

    @pl.when(k < _K)
    def _phase1():
        def accum(x):
            s = jnp.sum(x, axis=0, keepdims=True)
            q = jnp.sum(x * x, axis=0, keepdims=True)
            stash[pl.ds(k * _CHUNK, _CHUNK), :] = x

            @pl.when(k == 0)
            def _():
                acc_s[...] = s
                acc_q[...] = q

            @pl.when(k > 0)
            def _():
                acc_s[...] += s
                acc_q[...] += q

        @pl.when(j == 0)
        def _():
            accum(x0_ref[...])

        @pl.when(j == 1)
        def _():
            accum(x1_ref[...])

    @pl.when(k >= _K)
    def _phase2():
        kk = k - _K

        @pl.when(kk == 0)
        def _():
            def stats(g, be):
                mean = acc_s[...] * (1.0 / BATCH)
                var = acc_q[...] * (1.0 / BATCH) - mean * mean
                scale = g * lax.rsqrt(var + 1e-5)
                scale_r[...] = scale
                shift_r[...] = be - mean * scale

            @pl.when(j == 0)
            def _():
                stats(g0_ref[...], be0_ref[...])

            @pl.when(j == 1)
            def _():
                stats(g1_ref[...], be1_ref[...])

        x = stash[pl.ds(kk * _CHUNK, _CHUNK), :]
        h = x * scale_r[...] + shift_r[...]

        def emit(w, b):
            out_ref[0] = jnp.dot(h, w, preferred_element_type=jnp.float32) + b

        @pl.when(j == 0)
        def _():
            emit(w0_ref[...], b0_ref[...])

        @pl.when(j == 1)
        def _():
            emit(w1_ref[...], b1_ref[...])


def _paste_body(buf_ref, num_ref, out_ref):
    out_ref[...] = num_ref[...]


def _num_tc(x_num0, x_num1, gamma0, beta0, W0, b0, gamma1, beta1, W1, b1,
            interpret=False):
    small = lambda: pl.BlockSpec(memory_space=pltpu.MemorySpace.VMEM)
    return pl.pallas_call(
        _num_body,
        grid=(2, 2 * _K),
        in_specs=[
            pl.BlockSpec((_CHUNK, NUM_DIM),
                         lambda j, k: (jnp.where(j == 0, jnp.minimum(k, _K - 1),
                                                 _K - 1), 0)),
            pl.BlockSpec((_CHUNK, NUM_DIM),
                         lambda j, k: (jnp.where(j == 1, jnp.minimum(k, _K - 1),
                                                 0), 0)),
            small(), small(), small(), small(),
            small(), small(), small(), small(),
        ],
        out_specs=pl.BlockSpec(
            (1, _CHUNK, D_MODEL),
            lambda j, k: (j, jnp.maximum(k - _K, 0), 0)),
        out_shape=jax.ShapeDtypeStruct((2, BATCH, D_MODEL), jnp.float32),
        scratch_shapes=[
            pltpu.VMEM((BATCH, NUM_DIM), jnp.float32),
            pltpu.VMEM((1, NUM_DIM), jnp.float32),
            pltpu.VMEM((1, NUM_DIM), jnp.float32),
            pltpu.VMEM((1, NUM_DIM), jnp.float32),
            pltpu.VMEM((1, NUM_DIM), jnp.float32),
        ],
        interpret=interpret,
    )(x_num0, x_num1,
      gamma0.reshape(1, NUM_DIM), beta0.reshape(1, NUM_DIM), W0,
      b0.reshape(1, D_MODEL),
      gamma1.reshape(1, NUM_DIM), beta1.reshape(1, NUM_DIM), W1,
      b1.reshape(1, D_MODEL))


def kernel(x_cat0, x_cat1, x_cat2, x_cat3, x_num0, x_num1,
           table0, table1, table2, table3,
           gamma0, beta0, W0, b0, gamma1, beta1, W1, b1):
    buf = _gather_sc(x_cat0.astype(jnp.int32), x_cat1.astype(jnp.int32),
                     x_cat2.astype(jnp.int32), x_cat3.astype(jnp.int32),
                     table0, table1, table2, table3)

    num = _num_tc(x_num0, x_num1, gamma0, beta0, W0, b0,
                  gamma1, beta1, W1, b1)

    return pl.pallas_call(
        _paste_body,
        grid=(2,),
        in_specs=[
            pl.BlockSpec(memory_space=pltpu.MemorySpace.HBM),
            pl.BlockSpec((1, BATCH, D_MODEL), lambda j: (j, 0, 0)),
        ],
        out_specs=pl.BlockSpec((1, BATCH, D_MODEL), lambda j: (4 + j, 0, 0)),
        out_shape=jax.ShapeDtypeStruct((6, BATCH, D_MODEL), jnp.float32),
        input_output_aliases={0: 0},
    )(buf, num)


# speedup vs baseline: 1.0742x; 1.0742x over previous
import functools

import jax
import jax.numpy as jnp
from jax import lax
from jax.experimental import pallas as pl
from jax.experimental.pallas import tpu as pltpu
from jax.experimental.pallas import tpu_sc as plsc

D_MODEL = 128
BATCH = 4096
NUM_DIM = 512

_NC = 2
_NS = 16
_NW = _NC * _NS
_BPW = BATCH // _NW


def _gather_sc_inplace(i0, i1, i2, i3, t0, t1, t2, t3, out_ref):
    mesh = plsc.VectorSubcoreMesh(core_axis_name="c", subcore_axis_name="s")

    @functools.partial(
        pl.kernel,
        mesh=mesh,
        scratch_types=[
            pltpu.VMEM((4, _BPW), jnp.int32),
            pltpu.VMEM((4, _BPW, D_MODEL), jnp.float32),
            pltpu.SemaphoreType.DMA((4,)),
            pltpu.SemaphoreType.DMA((4,)),
            pltpu.SemaphoreType.DMA((4,)),
        ],
    )
    def body(ih0, ih1, ih2, ih3, tb0, tb1, tb2, tb3, out_hbm,
             idx_v, rows_v, sem_i, sem_g, sem_w):
        wid = lax.axis_index("s") * _NC + lax.axis_index("c")
        base = wid * _BPW
        iloads = []
        for i, ih in enumerate((ih0, ih1, ih2, ih3)):
            iloads.append(
                pltpu.async_copy(ih.at[pl.ds(base, _BPW)], idx_v.at[i],
                                 sem_i.at[i]))
        gathers = []
        for i, tbl in enumerate((tb0, tb1, tb2, tb3)):
            iloads[i].wait()
            gathers.append(
                pltpu.async_copy(tbl.at[idx_v.at[i]], rows_v.at[i], sem_g.at[i]))
        writes = []
        for i, g in enumerate(gathers):
            g.wait()
            writes.append(
                pltpu.async_copy(rows_v.at[i],
                                 out_hbm.at[i, pl.ds(base, _BPW)], sem_w.at[i]))
        for w in writes:
            w.wait()

    body(i0, i1, i2, i3, t0, t1, t2, t3, out_ref)


def _num_body(x0_ref, x1_ref, g0_ref, be0_ref, w0_ref, b0_ref,
              g1_ref, be1_ref, w1_ref, b1_ref, out_ref):
    j = pl.program_id(0)

    def compute(x, g, be, w, b):
        s = jnp.sum(x, axis=0, keepdims=True)
        q = jnp.sum(x * x, axis=0, keepdims=True)
        mean = s * (1.0 / BATCH)
        var = q * (1.0 / BATCH) - mean * mean
        scale = g * lax.rsqrt(var + 1e-5)
        shift = be - mean * scale
        h = x * scale + shift
        out_ref[0] = jnp.dot(h, w, preferred_element_type=jnp.float32) + b

    @pl.when(j == 0)
    def _():
        compute(x0_ref[...], g0_ref[...], be0_ref[...], w0_ref[...], b0_ref[...])

    @pl.when(j == 1)
    def _():
        compute(x1_ref[...], g1_ref[...], be1_ref[...], w1_ref[...], b1_ref[...])


def kernel(x_cat0, x_cat1, x_cat2, x_cat3, x_num0, x_num1,
           table0, table1, table2, table3,
           gamma0, beta0, W0, b0, gamma1, beta1, W1, b1):
    full = pl.BlockSpec(memory_space=pltpu.MemorySpace.VMEM)
    buf = pl.pallas_call(
        _num_body,
        grid=(2,),
        in_specs=[full] * 10,
        out_specs=pl.BlockSpec((1, BATCH, D_MODEL), lambda j: (4 + j, 0, 0)),
        out_shape=jax.ShapeDtypeStruct((6, BATCH, D_MODEL), jnp.float32),
    )(x_num0, x_num1,
      gamma0.reshape(1, NUM_DIM), beta0.reshape(1, NUM_DIM), W0,
      b0.reshape(1, D_MODEL),
      gamma1.reshape(1, NUM_DIM), beta1.reshape(1, NUM_DIM), W1,
      b1.reshape(1, D_MODEL))

    out_ref = jax.new_ref(buf)
    _gather_sc_inplace(x_cat0.astype(jnp.int32), x_cat1.astype(jnp.int32),
                       x_cat2.astype(jnp.int32), x_cat3.astype(jnp.int32),
                       table0, table1, table2, table3, out_ref)
    return jax.freeze(out_ref)


# speedup vs baseline: 1.1375x; 1.0590x over previous
import functools

import jax
import jax.numpy as jnp
from jax import lax
from jax.experimental import pallas as pl
from jax.experimental.pallas import tpu as pltpu
from jax.experimental.pallas import tpu_sc as plsc

D_MODEL = 128
BATCH = 4096
NUM_DIM = 512

_NC = 2
_NS = 16
_NW = _NC * _NS
_BPW = BATCH // _NW


def _gather_sc(i0, i1, i2, i3, t0, t1, t2, t3):
    mesh = plsc.VectorSubcoreMesh(core_axis_name="c", subcore_axis_name="s")

    @functools.partial(
        pl.kernel,
        mesh=mesh,
        out_type=jax.ShapeDtypeStruct((6, BATCH, D_MODEL), jnp.float32),
        scratch_types=[
            pltpu.VMEM((4, _BPW), jnp.int32),
            pltpu.VMEM((4, _BPW, D_MODEL), jnp.float32),
            pltpu.SemaphoreType.DMA((4,)),
            pltpu.SemaphoreType.DMA((4,)),
            pltpu.SemaphoreType.DMA((4,)),
        ],
    )
    def body(ih0, ih1, ih2, ih3, tb0, tb1, tb2, tb3, out_hbm,
             idx_v, rows_v, sem_i, sem_g, sem_w):
        wid = lax.axis_index("s") * _NC + lax.axis_index("c")
        base = wid * _BPW
        iloads = []
        for i, ih in enumerate((ih0, ih1, ih2, ih3)):
            iloads.append(
                pltpu.async_copy(ih.at[pl.ds(base, _BPW)], idx_v.at[i],
                                 sem_i.at[i]))
        gathers = []
        for i, tbl in enumerate((tb0, tb1, tb2, tb3)):
            iloads[i].wait()
            gathers.append(
                pltpu.async_copy(tbl.at[idx_v.at[i]], rows_v.at[i], sem_g.at[i]))
        writes = []
        for i, g in enumerate(gathers):
            g.wait()
            writes.append(
                pltpu.async_copy(rows_v.at[i],
                                 out_hbm.at[i, pl.ds(base, _BPW)], sem_w.at[i]))
        for w in writes:
            w.wait()

    return body(i0, i1, i2, i3, t0, t1, t2, t3)


def _paste_body(buf_ref, num_ref, out_ref):
    out_ref[...] = num_ref[...]


def _num_body(x0_ref, x1_ref, g0_ref, be0_ref, w0_ref, b0_ref,
              g1_ref, be1_ref, w1_ref, b1_ref, out_ref):
    j = pl.program_id(0)

    def compute(x, g, be, w, b):
        s = jnp.sum(x, axis=0, keepdims=True)
        q = jnp.sum(x * x, axis=0, keepdims=True)
        mean = s * (1.0 / BATCH)
        var = q * (1.0 / BATCH) - mean * mean
        scale = g * lax.rsqrt(var + 1e-5)
        shift = be - mean * scale
        h = x * scale + shift
        out_ref[0] = jnp.dot(h, w, preferred_element_type=jnp.float32) + b

    @pl.when(j == 0)
    def _():
        compute(x0_ref[...], g0_ref[...], be0_ref[...], w0_ref[...], b0_ref[...])

    @pl.when(j == 1)
    def _():
        compute(x1_ref[...], g1_ref[...], be1_ref[...], w1_ref[...], b1_ref[...])


def kernel(x_cat0, x_cat1, x_cat2, x_cat3, x_num0, x_num1,
           table0, table1, table2, table3,
           gamma0, beta0, W0, b0, gamma1, beta1, W1, b1):
    buf = _gather_sc(x_cat0.astype(jnp.int32), x_cat1.astype(jnp.int32),
                     x_cat2.astype(jnp.int32), x_cat3.astype(jnp.int32),
                     table0, table1, table2, table3)

    full = pl.BlockSpec(memory_space=pltpu.MemorySpace.VMEM)
    num = pl.pallas_call(
        _num_body,
        grid=(2,),
        in_specs=[full] * 10,
        out_specs=pl.BlockSpec((1, BATCH, D_MODEL), lambda j: (j, 0, 0)),
        out_shape=jax.ShapeDtypeStruct((2, BATCH, D_MODEL), jnp.float32),
    )(x_num0, x_num1,
      gamma0.reshape(1, NUM_DIM), beta0.reshape(1, NUM_DIM), W0,
      b0.reshape(1, D_MODEL),
      gamma1.reshape(1, NUM_DIM), beta1.reshape(1, NUM_DIM), W1,
      b1.reshape(1, D_MODEL))

    quarter = BATCH // 2
    return pl.pallas_call(
        _paste_body,
        grid=(2, 2),
        in_specs=[
            pl.BlockSpec(memory_space=pltpu.MemorySpace.HBM),
            pl.BlockSpec((1, quarter, D_MODEL), lambda j, c: (j, c, 0)),
        ],
        out_specs=pl.BlockSpec((1, quarter, D_MODEL),
                               lambda j, c: (4 + j, c, 0)),
        out_shape=jax.ShapeDtypeStruct((6, BATCH, D_MODEL), jnp.float32),
        input_output_aliases={0: 0},
    )(buf, num)
